# Initial kernel scaffold; baseline (speedup 1.0000x reference)
import jax
import jax.numpy as jnp
from jax.experimental import pallas as pl


def kernel(X, edge_index, W1, b1, scale1, W2, b2, scale2, gn_weight, gn_bias, gn_mean_scale):
    raise NotImplementedError("write your pallas kernel here")


# speedup vs baseline: 12.3984x; 12.3984x over previous
import functools

import jax
import jax.numpy as jnp
from jax import lax
from jax.experimental import pallas as pl
from jax.experimental.pallas import tpu as pltpu
from jax.experimental.pallas import tpu_sc as plsc

N_NODES = 10000
D = 128
N_EDGES = 320000

NC, NS = 2, 16
NW = NC * NS
CHUNK = 128
K = -(-N_EDGES // (NW * CHUNK))
E_PAD = NW * K * CHUNK
N_PAD = 10240
STRIPE = N_PAD // NS

NB = 2000
GRID = N_NODES // NB

def _sc_mesh():
    return plsc.VectorSubcoreMesh(
        core_axis_name="c", subcore_axis_name="s", num_cores=NC, num_subcores=NS)


def _gelu(x):
    return 0.5 * x * (1.0 + lax.erf(x * 0.7071067811865476))


def _sc_degree(cidx):
    @functools.partial(
        pl.kernel,
        mesh=_sc_mesh(),
        out_type=jax.ShapeDtypeStruct((NC, N_PAD), jnp.float32),
        scratch_types=[
            pltpu.VMEM((K, CHUNK), jnp.int32),
            pltpu.VMEM((CHUNK,), jnp.float32),
            pltpu.VMEM((CHUNK,), jnp.float32),
            pltpu.VMEM_SHARED((N_PAD,), jnp.float32),
        ],
    )
    def deg_kernel(cidx_hbm, degp_hbm, cidx_v, ones_v, zeros_v, acc):
        cid = lax.axis_index("c")
        sid = lax.axis_index("s")
        t = cid * NS + sid
        pltpu.sync_copy(cidx_hbm.at[t], cidx_v)
        for i in range(CHUNK // 16):
            ones_v[pl.ds(i * 16, 16)] = jnp.full((16,), 1.0, jnp.float32)
            zeros_v[pl.ds(i * 16, 16)] = jnp.zeros((16,), jnp.float32)
        for b in range(STRIPE // CHUNK):
            pltpu.sync_copy(zeros_v, acc.at[pl.ds(sid * STRIPE + b * CHUNK, CHUNK)])
        plsc.subcore_barrier()

        def body(j, carry):
            pltpu.sync_copy(ones_v, acc.at[cidx_v.at[j]], add=True)
            return carry

        lax.fori_loop(0, K, body, 0)
        plsc.subcore_barrier()
        for b in range(STRIPE // CHUNK):
            s = pl.ds(sid * STRIPE + b * CHUNK, CHUNK)
            pltpu.sync_copy(acc.at[s], degp_hbm.at[cid, s])

    return deg_kernel(cidx)


def _sc_conv(g, ridx, cidx):
    @functools.partial(
        pl.kernel,
        mesh=_sc_mesh(),
        out_type=jax.ShapeDtypeStruct((NC, N_PAD, D), jnp.float32),
        scratch_types=[
            pltpu.VMEM((K, CHUNK), jnp.int32),
            pltpu.VMEM((K, CHUNK), jnp.int32),
            pltpu.VMEM((CHUNK, D), jnp.float32),
            pltpu.VMEM_SHARED((N_PAD, D), jnp.float32),
            pltpu.SemaphoreType.DMA,
        ],
    )
    def conv_kernel(g_hbm, ridx_hbm, cidx_hbm, out_hbm,
                    ridx_v, cidx_v, rows_v, acc, sem):
        cid = lax.axis_index("c")
        sid = lax.axis_index("s")
        t = cid * NS + sid
        pltpu.sync_copy(ridx_hbm.at[t], ridx_v)
        pltpu.sync_copy(cidx_hbm.at[t], cidx_v)

        def zero_row(r, carry):
            for c8 in range(D // 16):
                rows_v[r, pl.ds(c8 * 16, 16)] = jnp.zeros((16,), jnp.float32)
            return carry

        lax.fori_loop(0, CHUNK, zero_row, 0)
        for b in range(STRIPE // CHUNK):
            pltpu.sync_copy(rows_v, acc.at[pl.ds(sid * STRIPE + b * CHUNK, CHUNK)])
        plsc.subcore_barrier()

        def body(j, carry):
            pltpu.async_copy(g_hbm.at[ridx_v.at[j]], rows_v, sem).wait()
            pltpu.sync_copy(rows_v, acc.at[cidx_v.at[j]], add=True)
            return carry

        lax.fori_loop(0, K, body, 0)
        plsc.subcore_barrier()
        for b in range(STRIPE // CHUNK):
            s = pl.ds(sid * STRIPE + b * CHUNK, CHUNK)
            pltpu.sync_copy(acc.at[s], out_hbm.at[cid, s])

    return conv_kernel(g, ridx, cidx)


def _tc_prep(X, W1, degt):
    NBP = 2048

    def body(x_ref, w1_ref, degt_ref, xt_ref, g1_ref, dis_ref):
        xb = x_ref[...]
        xt = xb.T
        db = degt_ref[...]
        deg = db[:, 0:1] + db[:, 1:2] + 1.0
        dis = lax.rsqrt(deg)
        h1 = lax.dot_general(xt, w1_ref[...], (((1,), (1,)), ((), ())),
                             preferred_element_type=jnp.float32)
        xt_ref[...] = xt
        g1_ref[...] = h1 * dis
        dis_ref[...] = jnp.broadcast_to(dis, (NBP, D))

    return pl.pallas_call(
        body,
        grid=(N_PAD // NBP,),
        in_specs=[
            pl.BlockSpec((D, NBP), lambda i: (0, i)),
            pl.BlockSpec((D, D), lambda i: (0, 0)),
            pl.BlockSpec((NBP, NC), lambda i: (i, 0)),
        ],
        out_specs=[pl.BlockSpec((NBP, D), lambda i: (i, 0))] * 3,
        out_shape=[jax.ShapeDtypeStruct((N_NODES, D), jnp.float32)] * 3,
    )(X, W1, degt)


def _tc_mid(s1, g1, dis, xt, W2, b1, scale1):
    def body(s_ref, g1_ref, dis_ref, xt_ref, w2_ref, b1_ref, sc1_ref,
             f_ref, g2_ref):
        sb = s_ref[...]
        s = sb[0] + sb[1]
        g1b = g1_ref[...]
        dis = dis_ref[...]
        xtb = xt_ref[...]
        c1 = dis * (s + g1b) + b1_ref[...]
        mn = jnp.sqrt(jnp.sum(c1 * c1, axis=1, keepdims=True))
        xn = jnp.sqrt(jnp.sum(xtb * xtb, axis=1, keepdims=True))
        f = _gelu(c1 / jnp.maximum(mn, 1e-12) * xn * sc1_ref[0, 0])
        h2 = lax.dot_general(f, w2_ref[...], (((1,), (1,)), ((), ())),
                             preferred_element_type=jnp.float32)
        f_ref[...] = f
        g2_ref[...] = h2 * dis

    return pl.pallas_call(
        body,
        grid=(GRID,),
        in_specs=[
            pl.BlockSpec((NC, NB, D), lambda i: (0, i, 0)),
            pl.BlockSpec((NB, D), lambda i: (i, 0)),
            pl.BlockSpec((NB, D), lambda i: (i, 0)),
            pl.BlockSpec((NB, D), lambda i: (i, 0)),
            pl.BlockSpec((D, D), lambda i: (0, 0)),
            pl.BlockSpec((1, D), lambda i: (0, 0)),
            pl.BlockSpec((1, 1), lambda i: (0, 0)),
        ],
        out_specs=[pl.BlockSpec((NB, D), lambda i: (i, 0))] * 2,
        out_shape=[jax.ShapeDtypeStruct((N_NODES, D), jnp.float32)] * 2,
    )(s1, g1, dis, xt, W2, b1, scale1)


def _tc_res(s2, g2, dis, f, xt, b2, scale2):
    def body(s_ref, g2_ref, dis_ref, f_ref, xt_ref, b2_ref, sc2_ref,
             y_ref, mom_ref):
        sb = s_ref[...]
        s = sb[0] + sb[1]
        g2b = g2_ref[...]
        dis = dis_ref[...]
        fb = f_ref[...]
        c2 = dis * (s + g2b) + b2_ref[...]
        mn = jnp.sqrt(jnp.sum(c2 * c2, axis=1, keepdims=True))
        fn = jnp.sqrt(jnp.sum(fb * fb, axis=1, keepdims=True))
        f2 = _gelu(c2 / jnp.maximum(mn, 1e-12) * fn * sc2_ref[0, 0])
        y = f2 + xt_ref[...]
        y_ref[...] = y
        s1m = jnp.sum(y, axis=0, keepdims=True)
        s2m = jnp.sum(y * y, axis=0, keepdims=True)
        mom_ref[...] = jnp.concatenate(
            [s1m, s2m, jnp.zeros((6, D), jnp.float32)], axis=0)[None]

    return pl.pallas_call(
        body,
        grid=(GRID,),
        in_specs=[
            pl.BlockSpec((NC, NB, D), lambda i: (0, i, 0)),
            pl.BlockSpec((NB, D), lambda i: (i, 0)),
            pl.BlockSpec((NB, D), lambda i: (i, 0)),
            pl.BlockSpec((NB, D), lambda i: (i, 0)),
            pl.BlockSpec((NB, D), lambda i: (i, 0)),
            pl.BlockSpec((1, D), lambda i: (0, 0)),
            pl.BlockSpec((1, 1), lambda i: (0, 0)),
        ],
        out_specs=[
            pl.BlockSpec((NB, D), lambda i: (i, 0)),
            pl.BlockSpec((1, 8, D), lambda i: (i, 0, 0)),
        ],
        out_shape=[
            jax.ShapeDtypeStruct((N_NODES, D), jnp.float32),
            jax.ShapeDtypeStruct((GRID, 8, D), jnp.float32),
        ],
    )(s2, g2, dis, f, xt, b2, scale2)


def _tc_final(y, mom, gn_weight, gn_bias, gn_mean_scale):
    def body(y_ref, mom_ref, w_ref, b_ref, ms_ref, out_ref):
        mom = mom_ref[...]
        s1m = jnp.sum(mom[:, 0, :], axis=0, keepdims=True)
        s2m = jnp.sum(mom[:, 1, :], axis=0, keepdims=True)
        inv_n = 1.0 / N_NODES
        mean = s1m * inv_n
        u = mean * ms_ref[...]
        var = s2m * inv_n - 2.0 * u * mean + u * u
        inv = lax.rsqrt(var + 1e-5)
        yb = y_ref[...]
        out_ref[...] = _gelu((yb - u) * inv * w_ref[...] + b_ref[...])

    return pl.pallas_call(
        body,
        grid=(GRID,),
        in_specs=[
            pl.BlockSpec((NB, D), lambda i: (i, 0)),
            pl.BlockSpec((GRID, 8, D), lambda i: (0, 0, 0)),
            pl.BlockSpec((1, D), lambda i: (0, 0)),
            pl.BlockSpec((1, D), lambda i: (0, 0)),
            pl.BlockSpec((1, D), lambda i: (0, 0)),
        ],
        out_specs=pl.BlockSpec((NB, D), lambda i: (i, 0)),
        out_shape=jax.ShapeDtypeStruct((N_NODES, D), jnp.float32),
    )(y, mom, gn_weight, gn_bias, gn_mean_scale)


def kernel(X, edge_index, W1, b1, scale1, W2, b2, scale2,
           gn_weight, gn_bias, gn_mean_scale):
    ei = edge_index.astype(jnp.int32)
    pad = E_PAD - N_EDGES
    ridx = jnp.concatenate(
        [ei[0], jnp.zeros((pad,), jnp.int32)]).reshape(NW, K, CHUNK)
    cidx = jnp.concatenate(
        [ei[1], jnp.full((pad,), N_NODES, jnp.int32)]).reshape(NW, K, CHUNK)

    degp = _sc_degree(cidx)
    degt = degp.T

    xt, g1, dis = _tc_prep(X, W1, degt)
    s1 = _sc_conv(g1, ridx, cidx)
    f, g2 = _tc_mid(s1, g1, dis, xt, W2,
                    b1.reshape(1, D), scale1.reshape(1, 1))
    s2 = _sc_conv(g2, ridx, cidx)
    y, mom = _tc_res(s2, g2, dis, f, xt,
                     b2.reshape(1, D), scale2.reshape(1, 1))
    return _tc_final(y, mom, gn_weight.reshape(1, D), gn_bias.reshape(1, D),
                     gn_mean_scale.reshape(1, D))


# speedup vs baseline: 24.0546x; 1.9401x over previous
import functools

import jax
import jax.numpy as jnp
from jax import lax
from jax.experimental import pallas as pl
from jax.experimental.pallas import tpu as pltpu
from jax.experimental.pallas import tpu_sc as plsc

N_NODES = 10000
D = 128
N_EDGES = 320000

NC, NS = 2, 16
NW = NC * NS
CHUNK = 128
NBUF = 2
IDXG = 40
K = -(-(-(-N_EDGES // (NW * CHUNK))) // IDXG) * IDXG
E_PAD = NW * K * CHUNK
N_PAD = 10240
STRIPE = N_PAD // NS

NB = 2000
GRID = N_NODES // NB

def _sc_mesh():
    return plsc.VectorSubcoreMesh(
        core_axis_name="c", subcore_axis_name="s", num_cores=NC, num_subcores=NS)


def _gelu(x):
    return 0.5 * x * (1.0 + lax.erf(x * 0.7071067811865476))


def _sc_degree(cidx):
    @functools.partial(
        pl.kernel,
        mesh=_sc_mesh(),
        out_type=jax.ShapeDtypeStruct((NC, N_PAD), jnp.float32),
        scratch_types=[
            pltpu.VMEM((K, CHUNK), jnp.int32),
            pltpu.VMEM((CHUNK,), jnp.float32),
            pltpu.VMEM((CHUNK,), jnp.float32),
            pltpu.VMEM_SHARED((N_PAD,), jnp.float32),
            pltpu.SemaphoreType.DMA,
        ],
    )
    def deg_kernel(cidx_hbm, degp_hbm, cidx_v, ones_v, zeros_v, acc, dsem):
        cid = lax.axis_index("c")
        sid = lax.axis_index("s")
        t = cid * NS + sid
        pltpu.sync_copy(cidx_hbm.at[t], cidx_v)
        for i in range(CHUNK // 16):
            ones_v[pl.ds(i * 16, 16)] = jnp.full((16,), 1.0, jnp.float32)
            zeros_v[pl.ds(i * 16, 16)] = jnp.zeros((16,), jnp.float32)
        for b in range(STRIPE // CHUNK):
            pltpu.sync_copy(zeros_v, acc.at[pl.ds(sid * STRIPE + b * CHUNK, CHUNK)])
        plsc.subcore_barrier()

        FK = 8

        def body(r8, carry):
            for q in range(FK):
                pltpu.async_copy(ones_v, acc.at[cidx_v.at[r8 * FK + q]],
                                 dsem, add=True)
            for q in range(FK):
                pltpu.make_async_copy(ones_v, acc.at[cidx_v.at[r8 * FK + q]],
                                      dsem).wait()
            return carry

        lax.fori_loop(0, K // FK, body, 0)
        plsc.subcore_barrier()
        for b in range(STRIPE // CHUNK):
            s = pl.ds(sid * STRIPE + b * CHUNK, CHUNK)
            pltpu.sync_copy(acc.at[s], degp_hbm.at[cid, s])

    return deg_kernel(cidx)


def _sc_conv(g, ridx, cidx):
    @functools.partial(
        pl.kernel,
        mesh=_sc_mesh(),
        out_type=jax.ShapeDtypeStruct((NC, N_PAD, D), jnp.float32),
        scratch_types=[
            pltpu.VMEM((IDXG, CHUNK), jnp.int32),
            pltpu.VMEM((IDXG, CHUNK), jnp.int32),
        ] + [pltpu.VMEM((CHUNK, D), jnp.float32)] * NBUF
          + [pltpu.VMEM_SHARED((N_PAD, D), jnp.float32)]
          + [pltpu.SemaphoreType.DMA] * (2 * NBUF),
    )
    def conv_kernel(g_hbm, ridx_hbm, cidx_hbm, out_hbm,
                    ridx_v, cidx_v, *rest):
        rows = rest[:NBUF]
        acc = rest[NBUF]
        gsems = rest[NBUF + 1:NBUF + 1 + NBUF]
        ssems = rest[NBUF + 1 + NBUF:]
        cid = lax.axis_index("c")
        sid = lax.axis_index("s")
        t = cid * NS + sid

        def zero_row(r, carry):
            for c8 in range(D // 16):
                rows[0][r, pl.ds(c8 * 16, 16)] = jnp.zeros((16,), jnp.float32)
            return carry

        lax.fori_loop(0, CHUNK, zero_row, 0)
        for b in range(STRIPE // CHUNK):
            pltpu.sync_copy(rows[0], acc.at[pl.ds(sid * STRIPE + b * CHUNK, CHUNK)])
        plsc.subcore_barrier()

        for grp in range(K // IDXG):
            pltpu.sync_copy(ridx_hbm.at[t, pl.ds(grp * IDXG, IDXG)], ridx_v)
            pltpu.sync_copy(cidx_hbm.at[t, pl.ds(grp * IDXG, IDXG)], cidx_v)
            for b in range(NBUF):
                pltpu.async_copy(g_hbm.at[ridx_v.at[b]], rows[b], gsems[b])

            def body(r, carry):
                for b in range(NBUF):
                    j = r * NBUF + b
                    pltpu.make_async_copy(g_hbm.at[ridx_v.at[j]], rows[b],
                                          gsems[b]).wait()
                    pltpu.async_copy(rows[b], acc.at[cidx_v.at[j]],
                                     ssems[b], add=True)
                for b in range(NBUF):
                    j = r * NBUF + b
                    pltpu.make_async_copy(rows[b], acc.at[cidx_v.at[j]],
                                          ssems[b]).wait()

                    @pl.when(j + NBUF < IDXG)
                    def _():
                        pltpu.async_copy(g_hbm.at[ridx_v.at[j + NBUF]],
                                         rows[b], gsems[b])
                return carry

            lax.fori_loop(0, IDXG // NBUF, body, 0)
        plsc.subcore_barrier()
        for b in range(STRIPE // CHUNK):
            s = pl.ds(sid * STRIPE + b * CHUNK, CHUNK)
            pltpu.sync_copy(acc.at[s], out_hbm.at[cid, s])

    return conv_kernel(g, ridx, cidx)


def _tc_prep(X, W1, degt):
    NBP = 2048

    def body(x_ref, w1_ref, degt_ref, xt_ref, g1_ref, dis_ref, xn_ref):
        xb = x_ref[...]
        xt = xb.T
        db = degt_ref[...]
        deg = db[:, 0:1] + db[:, 1:2] + 1.0
        dis = lax.rsqrt(deg)
        h1 = lax.dot_general(xt, w1_ref[...], (((1,), (1,)), ((), ())),
                             preferred_element_type=jnp.float32)
        xt_ref[...] = xt
        g1_ref[...] = h1 * dis
        dis_ref[...] = jnp.broadcast_to(dis, (NBP, 8))
        xn = jnp.sqrt(jnp.sum(xt * xt, axis=1, keepdims=True))
        xn_ref[...] = jnp.broadcast_to(xn, (NBP, 8))

    return pl.pallas_call(
        body,
        grid=(N_PAD // NBP,),
        in_specs=[
            pl.BlockSpec((D, NBP), lambda i: (0, i)),
            pl.BlockSpec((D, D), lambda i: (0, 0)),
            pl.BlockSpec((NBP, NC), lambda i: (i, 0)),
        ],
        out_specs=[pl.BlockSpec((NBP, D), lambda i: (i, 0))] * 2
                  + [pl.BlockSpec((NBP, 8), lambda i: (i, 0))] * 2,
        out_shape=[jax.ShapeDtypeStruct((N_NODES, D), jnp.float32)] * 2
                  + [jax.ShapeDtypeStruct((N_NODES, 8), jnp.float32)] * 2,
    )(X, W1, degt)


def _tc_mid(s1, g1, dis, xn, W2, b1, scale1):
    def body(s_ref, g1_ref, dis_ref, xn_ref, w2_ref, b1_ref, sc1_ref,
             f_ref, g2_ref, fn_ref):
        sb = s_ref[...]
        s = sb[0] + sb[1]
        g1b = g1_ref[...]
        dis = dis_ref[...][:, 0:1]
        xn = xn_ref[...][:, 0:1]
        c1 = dis * (s + g1b) + b1_ref[...]
        mn = jnp.sqrt(jnp.sum(c1 * c1, axis=1, keepdims=True))
        f = _gelu(c1 / jnp.maximum(mn, 1e-12) * (xn * sc1_ref[0, 0]))
        h2 = lax.dot_general(f, w2_ref[...], (((1,), (1,)), ((), ())),
                             preferred_element_type=jnp.float32)
        f_ref[...] = f
        g2_ref[...] = h2 * dis
        fn = jnp.sqrt(jnp.sum(f * f, axis=1, keepdims=True))
        fn_ref[...] = jnp.broadcast_to(fn, (NB, 8))

    return pl.pallas_call(
        body,
        grid=(GRID,),
        in_specs=[
            pl.BlockSpec((NC, NB, D), lambda i: (0, i, 0)),
            pl.BlockSpec((NB, D), lambda i: (i, 0)),
            pl.BlockSpec((NB, 8), lambda i: (i, 0)),
            pl.BlockSpec((NB, 8), lambda i: (i, 0)),
            pl.BlockSpec((D, D), lambda i: (0, 0)),
            pl.BlockSpec((1, D), lambda i: (0, 0)),
            pl.BlockSpec((1, 1), lambda i: (0, 0)),
        ],
        out_specs=[pl.BlockSpec((NB, D), lambda i: (i, 0))] * 2
                  + [pl.BlockSpec((NB, 8), lambda i: (i, 0))],
        out_shape=[jax.ShapeDtypeStruct((N_NODES, D), jnp.float32)] * 2
                  + [jax.ShapeDtypeStruct((N_NODES, 8), jnp.float32)],
    )(s1, g1, dis, xn, W2, b1, scale1)


def _tc_resfinal(s2, g2, dis, fn, xt, b2, scale2, gnw, gnb, gnms):
    def body(s_ref, g2_ref, dis_ref, fn_ref, xt_ref, b2_ref, sc2_ref,
             w_ref, bb_ref, ms_ref, out_ref, y_s, mom_s):
        p = pl.program_id(0)
        i = pl.program_id(1)

        @pl.when(p == 0)
        def _():
            sb = s_ref[...]
            s = sb[0] + sb[1]
            dis = dis_ref[...][:, 0:1]
            fn = fn_ref[...][:, 0:1]
            c2 = dis * (s + g2_ref[...]) + b2_ref[...]
            mn = jnp.sqrt(jnp.sum(c2 * c2, axis=1, keepdims=True))
            f2 = _gelu(c2 / jnp.maximum(mn, 1e-12) * (fn * sc2_ref[0, 0]))
            y = f2 + xt_ref[...]
            y_s[pl.ds(i * NB, NB), :] = y
            s1m = jnp.sum(y, axis=0, keepdims=True)
            s2m = jnp.sum(y * y, axis=0, keepdims=True)

            @pl.when(i == 0)
            def _():
                mom_s[0:1, :] = s1m
                mom_s[1:2, :] = s2m

            @pl.when(i > 0)
            def _():
                mom_s[0:1, :] = mom_s[0:1, :] + s1m
                mom_s[1:2, :] = mom_s[1:2, :] + s2m

            out_ref[...] = y

        @pl.when(p == 1)
        def _():
            inv_n = 1.0 / N_NODES
            mean = mom_s[0:1, :] * inv_n
            u = mean * ms_ref[...]
            var = mom_s[1:2, :] * inv_n - 2.0 * u * mean + u * u
            inv = lax.rsqrt(var + 1e-5)
            yb = y_s[pl.ds(i * NB, NB), :]
            out_ref[...] = _gelu((yb - u) * inv * w_ref[...] + bb_ref[...])

    return pl.pallas_call(
        body,
        grid=(2, GRID),
        in_specs=[
            pl.BlockSpec((NC, NB, D), lambda p, i: (0, i * (1 - p), 0)),
            pl.BlockSpec((NB, D), lambda p, i: (i * (1 - p), 0)),
            pl.BlockSpec((NB, 8), lambda p, i: (i * (1 - p), 0)),
            pl.BlockSpec((NB, 8), lambda p, i: (i * (1 - p), 0)),
            pl.BlockSpec((NB, D), lambda p, i: (i * (1 - p), 0)),
            pl.BlockSpec((1, D), lambda p, i: (0, 0)),
            pl.BlockSpec((1, 1), lambda p, i: (0, 0)),
            pl.BlockSpec((1, D), lambda p, i: (0, 0)),
            pl.BlockSpec((1, D), lambda p, i: (0, 0)),
            pl.BlockSpec((1, D), lambda p, i: (0, 0)),
        ],
        out_specs=pl.BlockSpec((NB, D), lambda p, i: (i, 0)),
        out_shape=jax.ShapeDtypeStruct((N_NODES, D), jnp.float32),
        scratch_shapes=[
            pltpu.VMEM((N_NODES, D), jnp.float32),
            pltpu.VMEM((8, D), jnp.float32),
        ],
    )(s2, g2, dis, fn, xt, b2, scale2, gnw, gnb, gnms)


def kernel(X, edge_index, W1, b1, scale1, W2, b2, scale2,
           gn_weight, gn_bias, gn_mean_scale):
    ei = edge_index.astype(jnp.int32)
    pad = E_PAD - N_EDGES
    pad_r = jnp.arange(pad, dtype=jnp.int32) % N_NODES
    pad_c = N_NODES + jnp.arange(pad, dtype=jnp.int32) % (N_PAD - N_NODES)
    ridx = jnp.concatenate([ei[0], pad_r]).reshape(NW, K, CHUNK)
    cidx = jnp.concatenate([ei[1], pad_c]).reshape(NW, K, CHUNK)

    degp = _sc_degree(cidx)
    degt = degp.T

    xt, g1, dis, xn = _tc_prep(X, W1, degt)
    s1 = _sc_conv(g1, ridx, cidx)
    f, g2, fn = _tc_mid(s1, g1, dis, xn, W2,
                        b1.reshape(1, D), scale1.reshape(1, 1))
    s2 = _sc_conv(g2, ridx, cidx)
    return _tc_resfinal(s2, g2, dis, fn, xt,
                        b2.reshape(1, D), scale2.reshape(1, 1),
                        gn_weight.reshape(1, D), gn_bias.reshape(1, D),
                        gn_mean_scale.reshape(1, D))


# speedup vs baseline: 29.3192x; 1.2189x over previous
import functools

import jax
import jax.numpy as jnp
from jax import lax
from jax.experimental import pallas as pl
from jax.experimental.pallas import tpu as pltpu
from jax.experimental.pallas import tpu_sc as plsc

N_NODES = 10000
D = 128
N_EDGES = 320000

NC, NS = 2, 16
NW = NC * NS
CHUNK = 128
NBUF = 2
IDXG = 40
K = -(-(-(-N_EDGES // (NW * CHUNK))) // IDXG) * IDXG
E_PAD = NW * K * CHUNK
N_PAD = 10240
STRIPE = N_PAD // NS

NB = 2000
GRID = N_NODES // NB

def _sc_mesh():
    return plsc.VectorSubcoreMesh(
        core_axis_name="c", subcore_axis_name="s", num_cores=NC, num_subcores=NS)


def _gelu(x):
    return 0.5 * x * (1.0 + lax.erf(x * 0.7071067811865476))


def _sc_degree(cidx):
    @functools.partial(
        pl.kernel,
        mesh=_sc_mesh(),
        out_type=jax.ShapeDtypeStruct((NC, N_PAD), jnp.float32),
        scratch_types=[
            pltpu.VMEM((K, CHUNK), jnp.int32),
            pltpu.VMEM((CHUNK,), jnp.float32),
            pltpu.VMEM((CHUNK,), jnp.float32),
            pltpu.VMEM_SHARED((N_PAD,), jnp.float32),
        ],
    )
    def deg_kernel(cidx_hbm, degp_hbm, cidx_v, ones_v, zeros_v, acc):
        cid = lax.axis_index("c")
        sid = lax.axis_index("s")
        t = cid * NS + sid
        pltpu.sync_copy(cidx_hbm.at[t], cidx_v)
        for i in range(CHUNK // 16):
            ones_v[pl.ds(i * 16, 16)] = jnp.full((16,), 1.0, jnp.float32)
            zeros_v[pl.ds(i * 16, 16)] = jnp.zeros((16,), jnp.float32)
        for b in range(STRIPE // CHUNK):
            pltpu.sync_copy(zeros_v, acc.at[pl.ds(sid * STRIPE + b * CHUNK, CHUNK)])
        plsc.subcore_barrier()

        def body(j, carry):
            pltpu.sync_copy(ones_v, acc.at[cidx_v.at[j]], add=True)
            return carry

        lax.fori_loop(0, K, body, 0)
        plsc.subcore_barrier()
        for b in range(STRIPE // CHUNK):
            s = pl.ds(sid * STRIPE + b * CHUNK, CHUNK)
            pltpu.sync_copy(acc.at[s], degp_hbm.at[cid, s])

    return deg_kernel(cidx)


def _sc_conv(g, ridx, cidx):
    @functools.partial(
        pl.kernel,
        mesh=_sc_mesh(),
        out_type=jax.ShapeDtypeStruct((NC, N_PAD, D), jnp.float32),
        scratch_types=[
            pltpu.VMEM((IDXG, CHUNK), jnp.int32),
            pltpu.VMEM((IDXG, CHUNK), jnp.int32),
        ] + [pltpu.VMEM((CHUNK, D), jnp.float32)] * NBUF
          + [pltpu.VMEM_SHARED((N_PAD, D), jnp.float32)]
          + [pltpu.SemaphoreType.DMA] * NBUF,
    )
    def conv_kernel(g_hbm, ridx_hbm, cidx_hbm, out_hbm,
                    ridx_v, cidx_v, *rest):
        rows = rest[:NBUF]
        acc = rest[NBUF]
        sems = rest[NBUF + 1:]
        cid = lax.axis_index("c")
        sid = lax.axis_index("s")
        t = cid * NS + sid

        def zero_row(r, carry):
            for c8 in range(D // 16):
                rows[0][r, pl.ds(c8 * 16, 16)] = jnp.zeros((16,), jnp.float32)
            return carry

        lax.fori_loop(0, CHUNK, zero_row, 0)
        for b in range(STRIPE // CHUNK):
            pltpu.sync_copy(rows[0], acc.at[pl.ds(sid * STRIPE + b * CHUNK, CHUNK)])
        plsc.subcore_barrier()

        for grp in range(K // IDXG):
            pltpu.sync_copy(ridx_hbm.at[t, pl.ds(grp * IDXG, IDXG)], ridx_v)
            pltpu.sync_copy(cidx_hbm.at[t, pl.ds(grp * IDXG, IDXG)], cidx_v)
            for b in range(NBUF):
                pltpu.async_copy(g_hbm.at[ridx_v.at[b]], rows[b], sems[b])

            def body(r, carry):
                for b in range(NBUF):
                    j = r * NBUF + b
                    pltpu.make_async_copy(g_hbm.at[ridx_v.at[j]], rows[b],
                                          sems[b]).wait()
                    pltpu.sync_copy(rows[b], acc.at[cidx_v.at[j]], add=True)

                    @pl.when(j + NBUF < IDXG)
                    def _():
                        pltpu.async_copy(g_hbm.at[ridx_v.at[j + NBUF]],
                                         rows[b], sems[b])
                return carry

            lax.fori_loop(0, IDXG // NBUF, body, 0)
        plsc.subcore_barrier()
        for b in range(STRIPE // CHUNK):
            s = pl.ds(sid * STRIPE + b * CHUNK, CHUNK)
            pltpu.sync_copy(acc.at[s], out_hbm.at[cid, s])

    return conv_kernel(g, ridx, cidx)


def _tc_prep(X, W1, degt):
    NBP = 2048

    def body(x_ref, w1_ref, degt_ref, xt_ref, g1_ref, dis_ref, xn_ref):
        xb = x_ref[...]
        xt = xb.T
        db = degt_ref[...]
        deg = db[:, 0:1] + db[:, 1:2] + 1.0
        dis = lax.rsqrt(deg)
        h1 = lax.dot_general(xt, w1_ref[...], (((1,), (1,)), ((), ())),
                             preferred_element_type=jnp.float32)
        xt_ref[...] = xt
        g1_ref[...] = h1 * dis
        dis_ref[...] = jnp.broadcast_to(dis, (NBP, 8))
        xn = jnp.sqrt(jnp.sum(xt * xt, axis=1, keepdims=True))
        xn_ref[...] = jnp.broadcast_to(xn, (NBP, 8))

    return pl.pallas_call(
        body,
        grid=(N_PAD // NBP,),
        in_specs=[
            pl.BlockSpec((D, NBP), lambda i: (0, i)),
            pl.BlockSpec((D, D), lambda i: (0, 0)),
            pl.BlockSpec((NBP, NC), lambda i: (i, 0)),
        ],
        out_specs=[pl.BlockSpec((NBP, D), lambda i: (i, 0))] * 2
                  + [pl.BlockSpec((NBP, 8), lambda i: (i, 0))] * 2,
        out_shape=[jax.ShapeDtypeStruct((N_NODES, D), jnp.float32)] * 2
                  + [jax.ShapeDtypeStruct((N_NODES, 8), jnp.float32)] * 2,
    )(X, W1, degt)


def _tc_mid(s1, g1, dis, xn, W2, b1, scale1):
    def body(s_ref, g1_ref, dis_ref, xn_ref, w2_ref, b1_ref, sc1_ref,
             f_ref, g2_ref, fn_ref):
        sb = s_ref[...]
        s = sb[0] + sb[1]
        g1b = g1_ref[...]
        dis = dis_ref[...][:, 0:1]
        xn = xn_ref[...][:, 0:1]
        c1 = dis * (s + g1b) + b1_ref[...]
        mn = jnp.sqrt(jnp.sum(c1 * c1, axis=1, keepdims=True))
        f = _gelu(c1 / jnp.maximum(mn, 1e-12) * (xn * sc1_ref[0, 0]))
        h2 = lax.dot_general(f, w2_ref[...], (((1,), (1,)), ((), ())),
                             preferred_element_type=jnp.float32)
        f_ref[...] = f
        g2_ref[...] = h2 * dis
        fn = jnp.sqrt(jnp.sum(f * f, axis=1, keepdims=True))
        fn_ref[...] = jnp.broadcast_to(fn, (NB, 8))

    return pl.pallas_call(
        body,
        grid=(GRID,),
        in_specs=[
            pl.BlockSpec((NC, NB, D), lambda i: (0, i, 0)),
            pl.BlockSpec((NB, D), lambda i: (i, 0)),
            pl.BlockSpec((NB, 8), lambda i: (i, 0)),
            pl.BlockSpec((NB, 8), lambda i: (i, 0)),
            pl.BlockSpec((D, D), lambda i: (0, 0)),
            pl.BlockSpec((1, D), lambda i: (0, 0)),
            pl.BlockSpec((1, 1), lambda i: (0, 0)),
        ],
        out_specs=[pl.BlockSpec((NB, D), lambda i: (i, 0))] * 2
                  + [pl.BlockSpec((NB, 8), lambda i: (i, 0))],
        out_shape=[jax.ShapeDtypeStruct((N_NODES, D), jnp.float32)] * 2
                  + [jax.ShapeDtypeStruct((N_NODES, 8), jnp.float32)],
    )(s1, g1, dis, xn, W2, b1, scale1)


def _tc_resfinal(s2, g2, dis, fn, xt, b2, scale2, gnw, gnb, gnms):
    def body(s_ref, g2_ref, dis_ref, fn_ref, xt_ref, b2_ref, sc2_ref,
             w_ref, bb_ref, ms_ref, out_ref, y_s, mom_s):
        p = pl.program_id(0)
        i = pl.program_id(1)

        @pl.when(p == 0)
        def _():
            sb = s_ref[...]
            s = sb[0] + sb[1]
            dis = dis_ref[...][:, 0:1]
            fn = fn_ref[...][:, 0:1]
            c2 = dis * (s + g2_ref[...]) + b2_ref[...]
            mn = jnp.sqrt(jnp.sum(c2 * c2, axis=1, keepdims=True))
            f2 = _gelu(c2 / jnp.maximum(mn, 1e-12) * (fn * sc2_ref[0, 0]))
            y = f2 + xt_ref[...]
            y_s[pl.ds(i * NB, NB), :] = y
            s1m = jnp.sum(y, axis=0, keepdims=True)
            s2m = jnp.sum(y * y, axis=0, keepdims=True)

            @pl.when(i == 0)
            def _():
                mom_s[0:1, :] = s1m
                mom_s[1:2, :] = s2m

            @pl.when(i > 0)
            def _():
                mom_s[0:1, :] = mom_s[0:1, :] + s1m
                mom_s[1:2, :] = mom_s[1:2, :] + s2m

            out_ref[...] = y

        @pl.when(p == 1)
        def _():
            inv_n = 1.0 / N_NODES
            mean = mom_s[0:1, :] * inv_n
            u = mean * ms_ref[...]
            var = mom_s[1:2, :] * inv_n - 2.0 * u * mean + u * u
            inv = lax.rsqrt(var + 1e-5)
            yb = y_s[pl.ds(i * NB, NB), :]
            out_ref[...] = _gelu((yb - u) * inv * w_ref[...] + bb_ref[...])

    return pl.pallas_call(
        body,
        grid=(2, GRID),
        in_specs=[
            pl.BlockSpec((NC, NB, D), lambda p, i: (0, i * (1 - p), 0)),
            pl.BlockSpec((NB, D), lambda p, i: (i * (1 - p), 0)),
            pl.BlockSpec((NB, 8), lambda p, i: (i * (1 - p), 0)),
            pl.BlockSpec((NB, 8), lambda p, i: (i * (1 - p), 0)),
            pl.BlockSpec((NB, D), lambda p, i: (i * (1 - p), 0)),
            pl.BlockSpec((1, D), lambda p, i: (0, 0)),
            pl.BlockSpec((1, 1), lambda p, i: (0, 0)),
            pl.BlockSpec((1, D), lambda p, i: (0, 0)),
            pl.BlockSpec((1, D), lambda p, i: (0, 0)),
            pl.BlockSpec((1, D), lambda p, i: (0, 0)),
        ],
        out_specs=pl.BlockSpec((NB, D), lambda p, i: (i, 0)),
        out_shape=jax.ShapeDtypeStruct((N_NODES, D), jnp.float32),
        scratch_shapes=[
            pltpu.VMEM((N_NODES, D), jnp.float32),
            pltpu.VMEM((8, D), jnp.float32),
        ],
    )(s2, g2, dis, fn, xt, b2, scale2, gnw, gnb, gnms)


def kernel(X, edge_index, W1, b1, scale1, W2, b2, scale2,
           gn_weight, gn_bias, gn_mean_scale):
    ei = edge_index.astype(jnp.int32)
    pad = E_PAD - N_EDGES
    pad_r = jnp.arange(pad, dtype=jnp.int32) % N_NODES
    pad_c = N_NODES + jnp.arange(pad, dtype=jnp.int32) % (N_PAD - N_NODES)
    ridx = jnp.concatenate([ei[0], pad_r]).reshape(NW, K, CHUNK)
    cidx = jnp.concatenate([ei[1], pad_c]).reshape(NW, K, CHUNK)

    degp = _sc_degree(cidx)
    degt = degp.T

    xt, g1, dis, xn = _tc_prep(X, W1, degt)
    s1 = _sc_conv(g1, ridx, cidx)
    f, g2, fn = _tc_mid(s1, g1, dis, xn, W2,
                        b1.reshape(1, D), scale1.reshape(1, 1))
    s2 = _sc_conv(g2, ridx, cidx)
    return _tc_resfinal(s2, g2, dis, fn, xt,
                        b2.reshape(1, D), scale2.reshape(1, 1),
                        gn_weight.reshape(1, D), gn_bias.reshape(1, D),
                        gn_mean_scale.reshape(1, D))
